# Initial kernel scaffold; baseline (speedup 1.0000x reference)
import jax
import jax.numpy as jnp
from jax.experimental import pallas as pl


def kernel(x, WLTag, W, b, emb_table):
    raise NotImplementedError("write your pallas kernel here")


# speedup vs baseline: 2.0700x; 2.0700x over previous
import functools

import jax
import jax.numpy as jnp
from jax import lax
from jax.experimental import pallas as pl
from jax.experimental.pallas import tpu as pltpu
from jax.experimental.pallas import tpu_sc as plsc

N = 100000
DIM_IN = 128
DIM_H = 96
DIM_PE = 32
NUM_TYPES = 1000

NW = 32
CH = 128
CPW = 25
N_PAD = NW * CPW * CH
LAST_FULL = N // CH - 1
TAIL_ROWS = N - (LAST_FULL + 1) * CH


def _sc_gather_body(idx_hbm, table_hbm, pe_hbm, idx_v, rows_v, sem):
    wid = lax.axis_index("s") * 2 + lax.axis_index("c")
    pltpu.sync_copy(idx_hbm.at[wid], idx_v)

    def chunk(j, carry):
        c = wid * CPW + j
        r0 = c * CH

        @pl.when(c <= LAST_FULL)
        def _full():
            pltpu.async_copy(table_hbm.at[idx_v.at[j]], rows_v, sem).wait()
            pltpu.sync_copy(rows_v, pe_hbm.at[pl.ds(r0, CH)])

        @pl.when(c == LAST_FULL + 1)
        def _tail():
            pltpu.async_copy(table_hbm.at[idx_v.at[j]], rows_v, sem).wait()
            pltpu.sync_copy(rows_v.at[pl.ds(0, TAIL_ROWS)],
                            pe_hbm.at[pl.ds((LAST_FULL + 1) * CH, TAIL_ROWS)])

        return carry

    lax.fori_loop(0, CPW, chunk, 0)


@functools.cache
def _sc_gather():
    return pl.kernel(
        _sc_gather_body,
        out_type=jax.ShapeDtypeStruct((N, DIM_PE), jnp.float32),
        mesh=plsc.VectorSubcoreMesh(core_axis_name="c", subcore_axis_name="s"),
        scratch_types=[
            pltpu.VMEM((CPW, CH), jnp.int32),
            pltpu.VMEM((CH, DIM_PE), jnp.float32),
            pltpu.SemaphoreType.DMA,
        ],
        compiler_params=pltpu.CompilerParams(use_tc_tiling_on_sc=False),
    )


def _tc_body(x_ref, w_ref, b_ref, pe_ref, out_ref):
    h = jnp.dot(x_ref[:], w_ref[:], preferred_element_type=jnp.float32)
    h = h + b_ref[:]
    out_ref[:] = jnp.concatenate([h, pe_ref[:]], axis=-1)


BLK = 2000


def _tc_matmul_concat(x, W, b2, pe):
    return pl.pallas_call(
        _tc_body,
        grid=(N // BLK,),
        in_specs=[
            pl.BlockSpec((BLK, DIM_IN), lambda i: (i, 0)),
            pl.BlockSpec((DIM_IN, DIM_H), lambda i: (0, 0)),
            pl.BlockSpec((1, DIM_H), lambda i: (0, 0)),
            pl.BlockSpec((BLK, DIM_PE), lambda i: (i, 0)),
        ],
        out_specs=pl.BlockSpec((BLK, DIM_IN), lambda i: (i, 0)),
        out_shape=jax.ShapeDtypeStruct((N, DIM_IN), jnp.float32),
        compiler_params=pltpu.CompilerParams(
            dimension_semantics=("parallel",),
        ),
    )(x, W, b2, pe)


def kernel(x, WLTag, W, b, emb_table):
    idx = WLTag.reshape(-1).astype(jnp.int32)
    idx = jnp.pad(idx, (0, N_PAD - N)).reshape(NW, CPW, CH)
    pe = _sc_gather()(idx, emb_table)
    return _tc_matmul_concat(x, W, b.reshape(1, DIM_H), pe)


# speedup vs baseline: 2.2756x; 1.0993x over previous
import functools

import jax
import jax.numpy as jnp
from jax import lax
from jax.experimental import pallas as pl
from jax.experimental.pallas import tpu as pltpu
from jax.experimental.pallas import tpu_sc as plsc

N = 100000
DIM_IN = 128
DIM_H = 96
DIM_PE = 32
NUM_TYPES = 1000

NW = 32
CH = 128
CPW = 25
N_PAD = NW * CPW * CH
LAST_FULL = N // CH - 1
TAIL_ROWS = N - (LAST_FULL + 1) * CH


def _sc_gather_body(idx_hbm, table_hbm, pe_hbm, idx_v, rows_v, sem):
    wid = lax.axis_index("s") * 2 + lax.axis_index("c")
    pltpu.sync_copy(idx_hbm.at[wid], idx_v)

    def chunk(j, carry):
        c = wid * CPW + j
        r0 = c * CH

        @pl.when(c <= LAST_FULL)
        def _full():
            pltpu.async_copy(table_hbm.at[idx_v.at[j]], rows_v, sem).wait()
            pltpu.sync_copy(rows_v, pe_hbm.at[pl.ds(r0, CH)])

        @pl.when(c == LAST_FULL + 1)
        def _tail():
            pltpu.async_copy(table_hbm.at[idx_v.at[j]], rows_v, sem).wait()
            pltpu.sync_copy(rows_v.at[pl.ds(0, TAIL_ROWS)],
                            pe_hbm.at[pl.ds((LAST_FULL + 1) * CH, TAIL_ROWS)])

        return carry

    lax.fori_loop(0, CPW, chunk, 0)


@functools.cache
def _sc_gather():
    return pl.kernel(
        _sc_gather_body,
        out_type=jax.ShapeDtypeStruct((N, DIM_PE), jnp.float32),
        mesh=plsc.VectorSubcoreMesh(core_axis_name="c", subcore_axis_name="s"),
        scratch_types=[
            pltpu.VMEM((CPW, CH), jnp.int32),
            pltpu.VMEM((CH, DIM_PE), jnp.float32),
            pltpu.SemaphoreType.DMA,
        ],
        compiler_params=pltpu.CompilerParams(use_tc_tiling_on_sc=False),
    )


def _tc_body(x_ref, w_ref, b_ref, pe_ref, out_ref):
    h = jnp.dot(x_ref[:], w_ref[:], preferred_element_type=jnp.float32)
    h = h + b_ref[:]
    out_ref[:] = jnp.concatenate([h, pe_ref[:]], axis=-1)


BLK = 4000


def _tc_matmul_concat(x, W, b2, pe):
    return pl.pallas_call(
        _tc_body,
        grid=(N // BLK,),
        in_specs=[
            pl.BlockSpec((BLK, DIM_IN), lambda i: (i, 0)),
            pl.BlockSpec((DIM_IN, DIM_H), lambda i: (0, 0)),
            pl.BlockSpec((1, DIM_H), lambda i: (0, 0)),
            pl.BlockSpec((BLK, DIM_PE), lambda i: (i, 0)),
        ],
        out_specs=pl.BlockSpec((BLK, DIM_IN), lambda i: (i, 0)),
        out_shape=jax.ShapeDtypeStruct((N, DIM_IN), jnp.float32),
        compiler_params=pltpu.CompilerParams(
            dimension_semantics=("parallel",),
        ),
    )(x, W, b2, pe)


def kernel(x, WLTag, W, b, emb_table):
    idx = WLTag.reshape(-1).astype(jnp.int32)
    idx = jnp.pad(idx, (0, N_PAD - N)).reshape(NW, CPW, CH)
    pe = _sc_gather()(idx, emb_table)
    return _tc_matmul_concat(x, W, b.reshape(1, DIM_H), pe)


# speedup vs baseline: 2.3230x; 1.0208x over previous
import functools

import jax
import jax.numpy as jnp
from jax import lax
from jax.experimental import pallas as pl
from jax.experimental.pallas import tpu as pltpu
from jax.experimental.pallas import tpu_sc as plsc

N = 100000
DIM_IN = 128
DIM_H = 96
DIM_PE = 32
NUM_TYPES = 1000

NW = 32
CH = 128
CPW = 25
N_PAD = NW * CPW * CH
LAST_FULL = N // CH - 1
TAIL_ROWS = N - (LAST_FULL + 1) * CH


def _sc_gather_body(idx_hbm, table_hbm, pe_hbm, idx_v, rows_v, sem):
    wid = lax.axis_index("s") * 2 + lax.axis_index("c")
    pltpu.sync_copy(idx_hbm.at[wid], idx_v)

    def chunk(j, carry):
        c = wid * CPW + j
        r0 = c * CH

        @pl.when(c <= LAST_FULL)
        def _full():
            pltpu.async_copy(table_hbm.at[idx_v.at[j]], rows_v, sem).wait()
            pltpu.sync_copy(rows_v, pe_hbm.at[pl.ds(r0, CH)])

        @pl.when(c == LAST_FULL + 1)
        def _tail():
            pltpu.async_copy(table_hbm.at[idx_v.at[j]], rows_v, sem).wait()
            pltpu.sync_copy(rows_v.at[pl.ds(0, TAIL_ROWS)],
                            pe_hbm.at[pl.ds((LAST_FULL + 1) * CH, TAIL_ROWS)])

        return carry

    lax.fori_loop(0, CPW, chunk, 0)


@functools.cache
def _sc_gather():
    return pl.kernel(
        _sc_gather_body,
        out_type=jax.ShapeDtypeStruct((N, DIM_PE), jnp.float32),
        mesh=plsc.VectorSubcoreMesh(core_axis_name="c", subcore_axis_name="s"),
        scratch_types=[
            pltpu.VMEM((CPW, CH), jnp.int32),
            pltpu.VMEM((CH, DIM_PE), jnp.float32),
            pltpu.SemaphoreType.DMA,
        ],
        compiler_params=pltpu.CompilerParams(use_tc_tiling_on_sc=False),
    )


def _tc_body(x_ref, w_ref, b_ref, pe_ref, out_ref):
    h = jnp.dot(x_ref[:], w_ref[:], preferred_element_type=jnp.float32)
    h = h + b_ref[:]
    out_ref[:] = jnp.concatenate([h, pe_ref[:]], axis=-1)


BLK = 10000


def _tc_matmul_concat(x, W, b2, pe):
    return pl.pallas_call(
        _tc_body,
        grid=(N // BLK,),
        in_specs=[
            pl.BlockSpec((BLK, DIM_IN), lambda i: (i, 0)),
            pl.BlockSpec((DIM_IN, DIM_H), lambda i: (0, 0)),
            pl.BlockSpec((1, DIM_H), lambda i: (0, 0)),
            pl.BlockSpec((BLK, DIM_PE), lambda i: (i, 0)),
        ],
        out_specs=pl.BlockSpec((BLK, DIM_IN), lambda i: (i, 0)),
        out_shape=jax.ShapeDtypeStruct((N, DIM_IN), jnp.float32),
        compiler_params=pltpu.CompilerParams(
            dimension_semantics=("parallel",),
        ),
    )(x, W, b2, pe)


def kernel(x, WLTag, W, b, emb_table):
    idx = WLTag.reshape(-1).astype(jnp.int32)
    idx = jnp.pad(idx, (0, N_PAD - N)).reshape(NW, CPW, CH)
    pe = _sc_gather()(idx, emb_table)
    return _tc_matmul_concat(x, W, b.reshape(1, DIM_H), pe)


# speedup vs baseline: 3.4985x; 1.5061x over previous
import functools

import jax
import jax.numpy as jnp
from jax import lax
from jax.experimental import pallas as pl
from jax.experimental.pallas import tpu as pltpu
from jax.experimental.pallas import tpu_sc as plsc

N = 100000
DIM_IN = 128
DIM_H = 96
DIM_PE = 32
NUM_TYPES = 1000

NW = 32
CH = 128
CPW = 25
N_PAD = NW * CPW * CH
LAST_FULL = N // CH - 1
TAIL_ROWS = N - (LAST_FULL + 1) * CH


def _sc_scatter_body(idx_hbm, table_hbm, out_ref, idx_v, rows_v, sem):
    wid = lax.axis_index("s") * 2 + lax.axis_index("c")
    pltpu.sync_copy(idx_hbm.at[wid], idx_v)

    def chunk(j, carry):
        c = wid * CPW + j
        r0 = c * CH

        @pl.when(c <= LAST_FULL)
        def _full():
            pltpu.async_copy(table_hbm.at[idx_v.at[j]], rows_v, sem).wait()
            pltpu.sync_copy(rows_v,
                            out_ref.at[pl.ds(r0, CH), pl.ds(DIM_H, DIM_PE)])

        @pl.when(c == LAST_FULL + 1)
        def _tail():
            pltpu.async_copy(table_hbm.at[idx_v.at[j]], rows_v, sem).wait()
            pltpu.sync_copy(rows_v.at[pl.ds(0, TAIL_ROWS)],
                            out_ref.at[pl.ds((LAST_FULL + 1) * CH, TAIL_ROWS),
                                       pl.ds(DIM_H, DIM_PE)])

        return carry

    lax.fori_loop(0, CPW, chunk, 0)


@functools.cache
def _sc_scatter():
    return pl.kernel(
        _sc_scatter_body,
        out_type=(),
        mesh=plsc.VectorSubcoreMesh(core_axis_name="c", subcore_axis_name="s"),
        scratch_types=[
            pltpu.VMEM((CPW, CH), jnp.int32),
            pltpu.VMEM((CH, DIM_PE), jnp.float32),
            pltpu.SemaphoreType.DMA,
        ],
        compiler_params=pltpu.CompilerParams(use_tc_tiling_on_sc=False),
    )


def _tc_body(x_ref, w_ref, b_ref, out_ref):
    h = jnp.dot(x_ref[:], w_ref[:], preferred_element_type=jnp.float32)
    out_ref[:, 0:DIM_H] = h + b_ref[:]


BLK = 10000


def _tc_matmul(x, W, b2):
    return pl.pallas_call(
        _tc_body,
        grid=(N // BLK,),
        in_specs=[
            pl.BlockSpec((BLK, DIM_IN), lambda i: (i, 0)),
            pl.BlockSpec((DIM_IN, DIM_H), lambda i: (0, 0)),
            pl.BlockSpec((1, DIM_H), lambda i: (0, 0)),
        ],
        out_specs=pl.BlockSpec((BLK, DIM_IN), lambda i: (i, 0)),
        out_shape=jax.ShapeDtypeStruct((N, DIM_IN), jnp.float32),
        compiler_params=pltpu.CompilerParams(
            dimension_semantics=("parallel",),
        ),
    )(x, W, b2)


def kernel(x, WLTag, W, b, emb_table):
    idx = WLTag.reshape(-1).astype(jnp.int32)
    idx = jnp.pad(idx, (0, N_PAD - N)).reshape(NW, CPW, CH)
    out_h = _tc_matmul(x, W, b.reshape(1, DIM_H))
    out_ref = jax.new_ref(out_h)
    _sc_scatter()(idx, emb_table, out_ref)
    return jax.freeze(out_ref)
